# baseline (speedup 1.0000x reference)
import jax
import jax.numpy as jnp
from jax.experimental import pallas as pl
from jax.experimental.pallas import tpu as pltpu

NUM_EMB_ = 1024
DIM_ = 64
CC_ = 0.6
BATCH_BLOCK = 32


def _vq_block_kernel(xt_ref, embt_ref, w_ref, qt_ref, idx_ref, loss_ref):
    i = pl.program_id(0)
    ng = pl.num_programs(0)
    embt = embt_ref[...]
    w = w_ref[...]
    e2 = jnp.sum(embt * embt, axis=0, keepdims=True).reshape(NUM_EMB_, 1)
    acc = jnp.float32(0.0)
    for b in range(BATCH_BLOCK):
        xb = xt_ref[b]
        dT = jax.lax.dot_general(embt, xb, (((0,), (0,)), ((), ())),
                                 preferred_element_type=jnp.float32)
        x2 = jnp.sum(xb * xb, axis=0, keepdims=True)
        dist = (x2 + e2 - 2.0 * dT) * w
        idx = jnp.argmin(dist, axis=0)
        kio = jax.lax.broadcasted_iota(jnp.int32, dist.shape, 0)
        oh = (kio == idx[None, :]).astype(jnp.float32)
        qt = jax.lax.dot_general(embt, oh, (((1,), (0,)), ((), ())),
                                 preferred_element_type=jnp.float32)
        qt_ref[b] = qt
        idx_ref[b] = idx
        d = qt - xb
        acc = acc + jnp.sum(d * d)

    @pl.when(i == 0)
    def _init():
        loss_ref[...] = jnp.zeros((1, 1), jnp.float32)

    loss_ref[...] += jnp.full((1, 1), acc, jnp.float32)

    @pl.when(i == ng - 1)
    def _finalize():
        loss_ref[...] = loss_ref[...] * ((1.0 + CC_) / (BATCH_BLOCK * ng * 576 * DIM_))


def kernel(inputs, emb_weight, scaling):
    B, S, D = inputs.shape
    K = emb_weight.shape[0]
    G = B // BATCH_BLOCK
    xt = jnp.swapaxes(inputs, 1, 2)
    embt = emb_weight.T
    hr_values = jnp.linspace(40.0, 180.0, K)
    w = (1.0 + scaling * ((hr_values - 100.0) / 70.0)).reshape(K, 1)

    qt, idx, loss2 = pl.pallas_call(
        _vq_block_kernel,
        grid=(G,),
        in_specs=[
            pl.BlockSpec((BATCH_BLOCK, D, S), lambda i: (i, 0, 0)),
            pl.BlockSpec((D, K), lambda i: (0, 0)),
            pl.BlockSpec((K, 1), lambda i: (0, 0)),
        ],
        out_specs=[
            pl.BlockSpec((BATCH_BLOCK, D, S), lambda i: (i, 0, 0)),
            pl.BlockSpec((BATCH_BLOCK, S), lambda i: (i, 0)),
            pl.BlockSpec((1, 1), lambda i: (0, 0)),
        ],
        out_shape=[
            jax.ShapeDtypeStruct((B, D, S), jnp.float32),
            jax.ShapeDtypeStruct((B, S), jnp.int32),
            jax.ShapeDtypeStruct((1, 1), jnp.float32),
        ],
        compiler_params=pltpu.CompilerParams(
            dimension_semantics=("arbitrary",),
        ),
    )(xt, embt, w)

    loss = loss2[0, 0]
    quantized_st = jnp.swapaxes(qt, 1, 2)
    return (quantized_st, loss, idx)


# speedup vs baseline: 1.0281x; 1.0281x over previous
import jax
import jax.numpy as jnp
from jax.experimental import pallas as pl
from jax.experimental.pallas import tpu as pltpu

NUM_EMB_ = 1024
DIM_ = 64
CC_ = 0.6
BATCH_BLOCK = 16


def _vq_block_kernel(xt_ref, embt_ref, w_ref, qt_ref, idx_ref, loss_ref):
    i = pl.program_id(0)
    ng = pl.num_programs(0)
    embt = embt_ref[...]
    w = w_ref[...]
    e2 = jnp.sum(embt * embt, axis=0, keepdims=True).reshape(NUM_EMB_, 1)
    acc = jnp.float32(0.0)
    for b in range(BATCH_BLOCK):
        xb = xt_ref[b]
        dT = jax.lax.dot_general(embt, xb, (((0,), (0,)), ((), ())),
                                 preferred_element_type=jnp.float32)
        x2 = jnp.sum(xb * xb, axis=0, keepdims=True)
        dist = (x2 + e2 - 2.0 * dT) * w
        idx = jnp.argmin(dist, axis=0)
        kio = jax.lax.broadcasted_iota(jnp.int32, dist.shape, 0)
        oh = (kio == idx[None, :]).astype(jnp.float32)
        qt = jax.lax.dot_general(embt, oh, (((1,), (0,)), ((), ())),
                                 preferred_element_type=jnp.float32)
        qt_ref[b] = qt
        idx_ref[b] = idx
        d = qt - xb
        acc = acc + jnp.sum(d * d)

    @pl.when(i == 0)
    def _init():
        loss_ref[...] = jnp.zeros((1, 1), jnp.float32)

    loss_ref[...] += jnp.full((1, 1), acc, jnp.float32)

    @pl.when(i == ng - 1)
    def _finalize():
        loss_ref[...] = loss_ref[...] * ((1.0 + CC_) / (BATCH_BLOCK * ng * 576 * DIM_))


def kernel(inputs, emb_weight, scaling):
    B, S, D = inputs.shape
    K = emb_weight.shape[0]
    G = B // BATCH_BLOCK
    xt = jnp.swapaxes(inputs, 1, 2)
    embt = emb_weight.T
    hr_values = jnp.linspace(40.0, 180.0, K)
    w = (1.0 + scaling * ((hr_values - 100.0) / 70.0)).reshape(K, 1)

    qt, idx, loss2 = pl.pallas_call(
        _vq_block_kernel,
        grid=(G,),
        in_specs=[
            pl.BlockSpec((BATCH_BLOCK, D, S), lambda i: (i, 0, 0)),
            pl.BlockSpec((D, K), lambda i: (0, 0)),
            pl.BlockSpec((K, 1), lambda i: (0, 0)),
        ],
        out_specs=[
            pl.BlockSpec((BATCH_BLOCK, D, S), lambda i: (i, 0, 0)),
            pl.BlockSpec((BATCH_BLOCK, S), lambda i: (i, 0)),
            pl.BlockSpec((1, 1), lambda i: (0, 0)),
        ],
        out_shape=[
            jax.ShapeDtypeStruct((B, D, S), jnp.float32),
            jax.ShapeDtypeStruct((B, S), jnp.int32),
            jax.ShapeDtypeStruct((1, 1), jnp.float32),
        ],
        compiler_params=pltpu.CompilerParams(
            dimension_semantics=("arbitrary",),
        ),
    )(xt, embt, w)

    loss = loss2[0, 0]
    quantized_st = jnp.swapaxes(qt, 1, 2)
    return (quantized_st, loss, idx)
